# baseline (speedup 1.0000x reference)
import functools

import jax
import jax.numpy as jnp
from jax import lax
from jax.experimental import pallas as pl
from jax.experimental.pallas import tpu as pltpu
from jax.experimental.pallas import tpu_sc as plsc

_N = 16384
_ROWS = 64
_COLS = 1_000_000
_L = 16
_CHUNK = 128
_NW = 32


def _sc_gather_sum(flat, idx):
    mesh = plsc.VectorSubcoreMesh(core_axis_name="c", subcore_axis_name="s")

    @functools.partial(
        pl.kernel,
        mesh=mesh,
        out_type=jax.ShapeDtypeStruct((_NW, _L), jnp.float32),
        scratch_types=[
            pltpu.VMEM((_N,), jnp.int32),
            pltpu.VMEM((_N,), jnp.float32),
            pltpu.VMEM((_L,), jnp.float32),
            pltpu.SemaphoreType.DMA,
        ],
    )
    def body(flat_hbm, idx_hbm, out_hbm, idx_v, val_v, res_v, sem):
        wid = lax.axis_index("s") * 2 + lax.axis_index("c")
        pltpu.sync_copy(idx_hbm, idx_v)

        def add_offset(off):
            def _b(i, carry):
                sl = pl.ds(i * _L, _L)
                idx_v[sl] = idx_v[sl] + off
                return carry
            lax.fori_loop(0, _N // _L, _b, 0)

        def gather_row():
            def _g(c, carry):
                sl = pl.ds(c * _CHUNK, _CHUNK)
                pltpu.async_copy(flat_hbm.at[idx_v.at[sl]], val_v.at[sl], sem)
                return carry
            lax.fori_loop(0, _N // _CHUNK, _g, 0)
            pltpu.make_async_copy(flat_hbm.at[pl.ds(0, _N)], val_v, sem).wait()

        def sum_row():
            def _s(i, acc):
                return acc + val_v[pl.ds(i * _L, _L)]
            acc = lax.fori_loop(0, _N // _L, _s, jnp.zeros((_L,), jnp.float32))
            lane = lax.iota(jnp.int32, _L)
            dnums = lax.GatherDimensionNumbers(
                offset_dims=(), collapsed_slice_dims=(0,),
                start_index_map=(0,))
            for k in (8, 4, 2, 1):
                perm = (lane ^ k).reshape(_L, 1)
                acc = acc + lax.gather(
                    acc, perm, dnums, (1,),
                    mode=lax.GatherScatterMode.PROMISE_IN_BOUNDS)
            return acc

        add_offset(wid * 2 * _COLS)
        gather_row()
        s0 = sum_row()
        add_offset(_COLS)
        gather_row()
        s1 = sum_row()

        lane = lax.iota(jnp.int32, _L)
        res = jnp.where(lane == 0, s0, jnp.zeros((_L,), jnp.float32))
        res = jnp.where(lane == 1, s1, res)
        res_v[...] = res
        pltpu.sync_copy(res_v, out_hbm.at[wid])

    return body(flat, idx)


def kernel(scdata, inputs):
    idx = jax.random.randint(
        jax.random.key(1), (_N,), 0, scdata.shape[1] - 1, dtype=jnp.int32)
    idx = jnp.sort(idx)
    idx = idx + (jnp.asarray(inputs, dtype=jnp.int32) - jnp.int32(_N))
    flat = scdata.reshape(-1)
    part = _sc_gather_sum(flat, idx)
    return part[:, :2].reshape(_ROWS)


# speedup vs baseline: 21.3609x; 21.3609x over previous
import functools

import jax
import jax.numpy as jnp
from jax import lax
from jax.experimental import pallas as pl
from jax.experimental.pallas import tpu as pltpu
from jax.experimental.pallas import tpu_sc as plsc

_N = 16384
_ROWS = 64
_COLS = 1_000_000
_L = 16
_BLK_C = 4096
_GRID = 245
_W = _GRID * _BLK_C
_NT = 16
_PER_T = _N // _NT
_ZCH = 6272


def _sc_counts(idx):
    mesh = plsc.VectorSubcoreMesh(core_axis_name="c", subcore_axis_name="s")

    @functools.partial(
        pl.kernel,
        mesh=mesh,
        out_type=jax.ShapeDtypeStruct((_W,), jnp.float32),
        scratch_types=[
            pltpu.VMEM((_PER_T,), jnp.int32),
            pltpu.VMEM((_ZCH,), jnp.float32),
            pltpu.VMEM((128,), jnp.float32),
            pltpu.VMEM_SHARED((_W,), jnp.float32),
        ],
    )
    def body(idx_hbm, w_hbm, idx_v, zero_v, one_v, shared):
        core = lax.axis_index("c")
        tid = lax.axis_index("s")

        @pl.when(core == 0)
        def _():
            def fill(ref, n, val):
                def _f(i, carry):
                    ref[pl.ds(i * _L, _L)] = jnp.full((_L,), val, jnp.float32)
                    return carry
                lax.fori_loop(0, n // _L, _f, 0)

            fill(zero_v, _ZCH, 0.0)
            fill(one_v, 128, 1.0)
            base = tid * (_W // _NT)

            def zchunk(k, carry):
                pltpu.sync_copy(
                    zero_v, shared.at[pl.ds(base + k * _ZCH, _ZCH)])
                return carry
            lax.fori_loop(0, 10, zchunk, 0)
            pltpu.sync_copy(idx_hbm.at[pl.ds(tid * _PER_T, _PER_T)], idx_v)
            plsc.subcore_barrier()

            def scatter(c, carry):
                pltpu.sync_copy(
                    one_v, shared.at[idx_v.at[pl.ds(c * 128, 128)]], add=True)
                return carry
            lax.fori_loop(0, _PER_T // 128, scatter, 0)
            plsc.subcore_barrier()

            def wchunk(k, carry):
                sl = pl.ds(base + k * _ZCH, _ZCH)
                pltpu.sync_copy(shared.at[sl], w_hbm.at[sl])
                return carry
            lax.fori_loop(0, 10, wchunk, 0)

    return body(idx)


def _tc_matvec(scdata, w):
    def body(sc_ref, w_ref, out_ref):
        pid = pl.program_id(0)
        data = sc_ref[...]
        gcol = pid * _BLK_C + lax.broadcasted_iota(jnp.int32, (_ROWS, _BLK_C), 1)
        data = jnp.where(gcol < _COLS, data, jnp.float32(0))
        part = lax.dot_general(
            data, w_ref[...], (((1,), (0,)), ((), ())),
            preferred_element_type=jnp.float32)

        @pl.when(pid == 0)
        def _():
            out_ref[...] = jnp.zeros_like(out_ref)

        out_ref[...] += part

    return pl.pallas_call(
        body,
        grid=(_GRID,),
        in_specs=[
            pl.BlockSpec((_ROWS, _BLK_C), lambda i: (0, i)),
            pl.BlockSpec((_BLK_C,), lambda i: (i,)),
        ],
        out_specs=pl.BlockSpec((_ROWS,), lambda i: (0,)),
        out_shape=jax.ShapeDtypeStruct((_ROWS,), jnp.float32),
    )(scdata, w)


def kernel(scdata, inputs):
    idx = jax.random.randint(
        jax.random.key(1), (_N,), 0, scdata.shape[1] - 1, dtype=jnp.int32)
    idx = idx + (jnp.asarray(inputs, dtype=jnp.int32) - jnp.int32(_N))
    w = _sc_counts(idx)
    return _tc_matvec(scdata, w)


# speedup vs baseline: 40.8075x; 1.9104x over previous
import functools

import jax
import jax.numpy as jnp
from jax import lax
from jax.experimental import pallas as pl
from jax.experimental.pallas import tpu as pltpu
from jax.experimental.pallas import tpu_sc as plsc

_N = 16384
_ROWS = 64
_COLS = 1_000_000
_L = 16
_BLK_C = 16384
_GRID = 62
_W = _GRID * _BLK_C
_NT = 16
_PER_T = _N // _NT
_ZCH = 7936


def _sc_counts(idx):
    mesh = plsc.VectorSubcoreMesh(core_axis_name="c", subcore_axis_name="s")

    @functools.partial(
        pl.kernel,
        mesh=mesh,
        out_type=jax.ShapeDtypeStruct((_W,), jnp.float32),
        scratch_types=[
            pltpu.VMEM((_PER_T,), jnp.int32),
            pltpu.VMEM((_ZCH,), jnp.float32),
            pltpu.VMEM((128,), jnp.float32),
            pltpu.VMEM_SHARED((_W,), jnp.float32),
        ],
    )
    def body(idx_hbm, w_hbm, idx_v, zero_v, one_v, shared):
        core = lax.axis_index("c")
        tid = lax.axis_index("s")

        @pl.when(core == 0)
        def _():
            def fill(ref, n, val):
                def _f(i, carry):
                    ref[pl.ds(i * _L, _L)] = jnp.full((_L,), val, jnp.float32)
                    return carry
                lax.fori_loop(0, n // _L, _f, 0)

            fill(zero_v, _ZCH, 0.0)
            fill(one_v, 128, 1.0)
            base = tid * (_W // _NT)

            def zchunk(k, carry):
                pltpu.sync_copy(
                    zero_v, shared.at[pl.ds(base + k * _ZCH, _ZCH)])
                return carry
            lax.fori_loop(0, 8, zchunk, 0)
            pltpu.sync_copy(idx_hbm.at[pl.ds(tid * _PER_T, _PER_T)], idx_v)
            plsc.subcore_barrier()

            def scatter(c, carry):
                pltpu.sync_copy(
                    one_v, shared.at[idx_v.at[pl.ds(c * 128, 128)]], add=True)
                return carry
            lax.fori_loop(0, _PER_T // 128, scatter, 0)
            plsc.subcore_barrier()

            def wchunk(k, carry):
                sl = pl.ds(base + k * _ZCH, _ZCH)
                pltpu.sync_copy(shared.at[sl], w_hbm.at[sl])
                return carry
            lax.fori_loop(0, 8, wchunk, 0)

    return body(idx)


def _tc_matvec(scdata, w):
    def body(sc_ref, w_ref, out_ref):
        pid = pl.program_id(0)
        part = lax.dot_general(
            sc_ref[...], w_ref[...], (((1,), (0,)), ((), ())),
            preferred_element_type=jnp.float32)

        @pl.when(pid == 0)
        def _():
            out_ref[...] = jnp.zeros_like(out_ref)

        out_ref[...] += part

    return pl.pallas_call(
        body,
        grid=(_GRID,),
        in_specs=[
            pl.BlockSpec((_ROWS, _BLK_C), lambda i: (0, i)),
            pl.BlockSpec((_BLK_C,), lambda i: (i,)),
        ],
        out_specs=pl.BlockSpec((_ROWS,), lambda i: (0,)),
        out_shape=jax.ShapeDtypeStruct((_ROWS,), jnp.float32),
    )(scdata, w)


def kernel(scdata, inputs):
    idx = jax.random.randint(
        jax.random.key(1), (_N,), 0, scdata.shape[1] - 1, dtype=jnp.int32)
    idx = idx + (jnp.asarray(inputs, dtype=jnp.int32) - jnp.int32(_N))
    w = _sc_counts(idx)
    return _tc_matvec(scdata, w)


# speedup vs baseline: 45.9190x; 1.1253x over previous
import functools

import jax
import jax.numpy as jnp
from jax import lax
from jax.experimental import pallas as pl
from jax.experimental.pallas import tpu as pltpu
from jax.experimental.pallas import tpu_sc as plsc

_N = 16384
_ROWS = 64
_COLS = 1_000_000
_L = 16
_BLK_C = 32768
_GRID = 31
_W = _GRID * _BLK_C
_NT = 16
_PER_T = _N // _NT
_ZCH = 7936


def _sc_counts(idx):
    mesh = plsc.VectorSubcoreMesh(core_axis_name="c", subcore_axis_name="s")

    @functools.partial(
        pl.kernel,
        mesh=mesh,
        out_type=jax.ShapeDtypeStruct((_W,), jnp.float32),
        scratch_types=[
            pltpu.VMEM((_PER_T,), jnp.int32),
            pltpu.VMEM((_ZCH,), jnp.float32),
            pltpu.VMEM((128,), jnp.float32),
            pltpu.VMEM_SHARED((_W,), jnp.float32),
        ],
    )
    def body(idx_hbm, w_hbm, idx_v, zero_v, one_v, shared):
        core = lax.axis_index("c")
        tid = lax.axis_index("s")

        @pl.when(core == 0)
        def _():
            def fill(ref, n, val):
                def _f(i, carry):
                    ref[pl.ds(i * _L, _L)] = jnp.full((_L,), val, jnp.float32)
                    return carry
                lax.fori_loop(0, n // _L, _f, 0)

            fill(zero_v, _ZCH, 0.0)
            fill(one_v, 128, 1.0)
            base = tid * (_W // _NT)

            def zchunk(k, carry):
                pltpu.sync_copy(
                    zero_v, shared.at[pl.ds(base + k * _ZCH, _ZCH)])
                return carry
            lax.fori_loop(0, 8, zchunk, 0)
            pltpu.sync_copy(idx_hbm.at[pl.ds(tid * _PER_T, _PER_T)], idx_v)
            plsc.subcore_barrier()

            def scatter(c, carry):
                pltpu.sync_copy(
                    one_v, shared.at[idx_v.at[pl.ds(c * 128, 128)]], add=True)
                return carry
            lax.fori_loop(0, _PER_T // 128, scatter, 0)
            plsc.subcore_barrier()

            def wchunk(k, carry):
                sl = pl.ds(base + k * _ZCH, _ZCH)
                pltpu.sync_copy(shared.at[sl], w_hbm.at[sl])
                return carry
            lax.fori_loop(0, 8, wchunk, 0)

    return body(idx)


def _tc_matvec(scdata, w):
    def body(sc_ref, w_ref, out_ref):
        pid = pl.program_id(0)
        part = lax.dot_general(
            sc_ref[...], w_ref[...], (((1,), (0,)), ((), ())),
            preferred_element_type=jnp.float32)

        @pl.when(pid == 0)
        def _():
            out_ref[...] = jnp.zeros_like(out_ref)

        out_ref[...] += part

    return pl.pallas_call(
        body,
        grid=(_GRID,),
        in_specs=[
            pl.BlockSpec((_ROWS, _BLK_C), lambda i: (0, i)),
            pl.BlockSpec((_BLK_C,), lambda i: (i,)),
        ],
        out_specs=pl.BlockSpec((_ROWS,), lambda i: (0,)),
        out_shape=jax.ShapeDtypeStruct((_ROWS,), jnp.float32),
    )(scdata, w)


def kernel(scdata, inputs):
    idx = jax.random.randint(
        jax.random.key(1), (_N,), 0, scdata.shape[1] - 1, dtype=jnp.int32)
    idx = idx + (jnp.asarray(inputs, dtype=jnp.int32) - jnp.int32(_N))
    w = _sc_counts(idx)
    return _tc_matvec(scdata, w)
